# Initial kernel scaffold; baseline (speedup 1.0000x reference)
import jax
import jax.numpy as jnp
from jax.experimental import pallas as pl


def kernel(task1_feat, task2_feat, W_g, b_g, W_s, b_s, grapharea, k):
    raise NotImplementedError("write your pallas kernel here")


# speedup vs baseline: 1.6692x; 1.6692x over previous
import functools

import jax
import jax.numpy as jnp
from jax import lax
from jax.experimental import pallas as pl
from jax.experimental.pallas import tpu as pltpu
from jax.experimental.pallas import tpu_sc as plsc

T = 256
D = 512
VG = 100000
VS = 16384
GA = 32
KTOP = 8
VB = 2048
NB = 49
VGP = NB * VB
SEGW = 128
NSEG = VGP // SEGW
SEG_PER_B = VB // SEGW
NEG = -1e30
BIGI = 2**30
EPS = 1e-8

_PREC = lax.Precision.DEFAULT


def _k1a_body(x_ref, w_ref, b_ref, raw_ref, segmax_ref, lse_ref, m_sc, s_sc):
    j = pl.program_id(0)
    x = x_ref[...]
    w = w_ref[...]
    l = lax.dot_general(x, w, (((1,), (0,)), ((), ())),
                        preferred_element_type=jnp.float32, precision=_PREC)
    l = l + b_ref[...]
    col = j * VB + lax.broadcasted_iota(jnp.int32, (1, VB), 1)
    l = jnp.where(col < VG, l, NEG)
    raw_ref[...] = l
    segmax_ref[...] = jnp.max(l.reshape(T, SEG_PER_B, SEGW),
                              axis=2).reshape(1, T, SEG_PER_B)

    @pl.when(j == 0)
    def _():
        m_sc[...] = jnp.full((T, 128), -3e38, jnp.float32)
        s_sc[...] = jnp.zeros((T, 128), jnp.float32)

    bmax = jnp.max(l, axis=1, keepdims=True)
    m_old = m_sc[:, 0:1]
    m_new = jnp.maximum(m_old, bmax)
    s_new = (s_sc[:, 0:1] * jnp.exp(m_old - m_new)
             + jnp.sum(jnp.exp(l - m_new), axis=1, keepdims=True))
    m_sc[...] = jnp.broadcast_to(m_new, (T, 128))
    s_sc[...] = jnp.broadcast_to(s_new, (T, 128))

    @pl.when(j == NB - 1)
    def _():
        lse_ref[...] = m_sc[...] + jnp.log(s_sc[...])


def _k1b_body(raw_ref, lse_ref, out_ref):
    out_ref[...] = raw_ref[...] - lse_ref[:, 0:1]


def _k2_body(segmax_ref, segids_ref):
    x = segmax_ref[...]
    col = lax.broadcasted_iota(jnp.int32, (T, NSEG), 1)
    ids = []
    for _ in range(KTOP):
        m = jnp.max(x, axis=1, keepdims=True)
        idx = jnp.min(jnp.where(x == m, col, BIGI), axis=1, keepdims=True)
        ids.append(idx)
        x = jnp.where(col == idx, -3e38, x)
    segids_ref[...] = jnp.concatenate(ids, axis=1)


def _k4_body(cands_ref, segids_ref, kidx_ref):
    segids = segids_ref[...]
    x = cands_ref[...]
    lane = lax.broadcasted_iota(jnp.int32, (T, SEGW), 1)
    gcol = jnp.concatenate(
        [segids[:, i:i + 1] * SEGW + lane for i in range(KTOP)], axis=1)
    ids = []
    for _ in range(KTOP):
        m = jnp.max(x, axis=1, keepdims=True)
        idx = jnp.min(jnp.where(x == m, gcol, BIGI), axis=1, keepdims=True)
        ids.append(idx)
        x = jnp.where(gcol == idx, -3e38, x)
    kidx_ref[...] = jnp.concatenate(ids, axis=1)


def _k6_body(x_ref, w_ref, b_ref, mask_ref, out_ref):
    j = pl.program_id(0)
    nblk = VS // VB
    x = x_ref[...]
    w = w_ref[...]
    l = lax.dot_general(x, w, (((1,), (0,)), ((), ())),
                        preferred_element_type=jnp.float32, precision=_PREC)
    l = l + b_ref[...]
    out_ref[:, pl.ds(j * VB, VB)] = jnp.where(mask_ref[...] > 0.5, l, NEG)

    @pl.when(j == nblk - 1)
    def _():
        m = jnp.full((T, 1), -3e38, jnp.float32)
        for i in range(nblk):
            m = jnp.maximum(
                m, jnp.max(out_ref[:, i * VB:(i + 1) * VB], axis=1,
                           keepdims=True))
        z = jnp.zeros((T, 1), jnp.float32)
        nsel = jnp.zeros((T, 1), jnp.float32)
        amax = jnp.full((T, 1), BIGI, jnp.int32)
        lane = lax.broadcasted_iota(jnp.int32, (T, VB), 1)
        for i in range(nblk):
            c = out_ref[:, i * VB:(i + 1) * VB]
            z = z + jnp.sum(jnp.exp(c - m), axis=1, keepdims=True)
            nsel = nsel + jnp.sum(jnp.where(c > -5e29, 1.0, 0.0), axis=1,
                                  keepdims=True)
            amax = jnp.minimum(
                amax,
                jnp.min(jnp.where(c == m, lane + i * VB, BIGI), axis=1,
                        keepdims=True))
        delta = EPS * (float(VS) - nsel)
        for i in range(nblk):
            c = out_ref[:, i * VB:(i + 1) * VB]
            p = jnp.exp(c - m) / z
            p = p - delta * jnp.where(lane + i * VB == amax, 1.0, 0.0)
            sel = c > -5e29
            out_ref[:, i * VB:(i + 1) * VB] = jnp.log(
                jnp.where(sel, p, EPS))


def _global_head(x, w_g, b_g):
    return pl.pallas_call(
        _k1a_body,
        grid=(NB,),
        in_specs=[
            pl.BlockSpec((T, D), lambda j: (0, 0)),
            pl.BlockSpec((D, VB), lambda j: (0, j)),
            pl.BlockSpec((1, VB), lambda j: (0, j)),
        ],
        out_specs=[
            pl.BlockSpec((T, VB), lambda j: (0, j)),
            pl.BlockSpec((1, T, SEG_PER_B), lambda j: (j, 0, 0)),
            pl.BlockSpec((T, 128), lambda j: (0, 0)),
        ],
        out_shape=[
            jax.ShapeDtypeStruct((T, VGP), jnp.float32),
            jax.ShapeDtypeStruct((NB, T, SEG_PER_B), jnp.float32),
            jax.ShapeDtypeStruct((T, 128), jnp.float32),
        ],
        scratch_shapes=[
            pltpu.VMEM((T, 128), jnp.float32),
            pltpu.VMEM((T, 128), jnp.float32),
        ],
    )(x, w_g, b_g)


def _shift_logits(raw_pad, lse):
    return pl.pallas_call(
        _k1b_body,
        grid=(NB,),
        in_specs=[
            pl.BlockSpec((T, VB), lambda j: (0, j)),
            pl.BlockSpec((T, 128), lambda j: (0, 0)),
        ],
        out_specs=pl.BlockSpec((T, VB), lambda j: (0, j)),
        out_shape=jax.ShapeDtypeStruct((T, VG), jnp.float32),
    )(raw_pad, lse)


def _top_segments(segmax):
    return pl.pallas_call(
        _k2_body,
        out_shape=jax.ShapeDtypeStruct((T, KTOP), jnp.int32),
    )(segmax)


def _top8_refine(cands, segids):
    return pl.pallas_call(
        _k4_body,
        out_shape=jax.ShapeDtypeStruct((T, KTOP), jnp.int32),
    )(cands, segids)


def _sense_head(x, w_s, b_s, mask):
    nblk = VS // VB
    return pl.pallas_call(
        _k6_body,
        grid=(nblk,),
        in_specs=[
            pl.BlockSpec((T, D), lambda j: (0, 0)),
            pl.BlockSpec((D, VB), lambda j: (0, j)),
            pl.BlockSpec((1, VB), lambda j: (0, j)),
            pl.BlockSpec((T, VB), lambda j: (0, j)),
        ],
        out_specs=pl.BlockSpec((T, VS), lambda j: (0, 0)),
        out_shape=jax.ShapeDtypeStruct((T, VS), jnp.float32),
    )(x, w_s, b_s, mask)


def _sc_gather(table, idx):
    B = idx.shape[0]
    W = table.shape[1]
    NW = 32
    bpw = B // NW
    mesh = plsc.VectorSubcoreMesh(core_axis_name="c", subcore_axis_name="s")

    @functools.partial(
        pl.kernel, mesh=mesh,
        out_type=jax.ShapeDtypeStruct((B, W), table.dtype),
        scratch_types=[
            pltpu.VMEM((bpw,), jnp.int32),
            pltpu.VMEM((bpw, W), table.dtype),
            pltpu.SemaphoreType.DMA,
        ],
    )
    def kfn(table_hbm, idx_hbm, out_hbm, idx_v, rows_v, sem):
        wid = lax.axis_index("s") * 2 + lax.axis_index("c")
        base = wid * bpw
        pltpu.sync_copy(idx_hbm.at[pl.ds(base, bpw)], idx_v)
        pltpu.async_copy(table_hbm.at[idx_v], rows_v, sem).wait()
        pltpu.sync_copy(rows_v, out_hbm.at[pl.ds(base, bpw)])

    return kfn(table, idx)


def _sc_mask(kflat, ga_packed):
    NW = 32
    rpw = (T * KTOP) // NW
    tpw = T // NW
    mesh = plsc.VectorSubcoreMesh(core_axis_name="c", subcore_axis_name="s")

    npair = tpw // 2

    @functools.partial(
        pl.kernel, mesh=mesh,
        out_type=jax.ShapeDtypeStruct((T, VS), jnp.float32),
        compiler_params=pltpu.CompilerParams(needs_layout_passes=False),
        scratch_types=[
            pltpu.VMEM((rpw,), jnp.int32),
            pltpu.VMEM((rpw,), jnp.int32),
            pltpu.VMEM((rpw, 4 * GA), jnp.int32),
            pltpu.VMEM((2, VS), jnp.float32),
            pltpu.VMEM((2, VS), jnp.float32),
            pltpu.SemaphoreType.DMA,
            pltpu.SemaphoreType.DMA,
        ],
    )
    def kfn(kidx_hbm, ga_hbm, mask_hbm, idx_v, q_v, rows_v, mrow0, mrow1,
            sem0, sem1):
        wid = lax.axis_index("s") * 2 + lax.axis_index("c")
        base = wid * rpw
        pltpu.sync_copy(kidx_hbm.at[pl.ds(base, rpw)], idx_v)
        for g in range(rpw // 16):
            v = idx_v[pl.ds(g * 16, 16)]
            q_v[pl.ds(g * 16, 16)] = (v & 3) * GA
            idx_v[pl.ds(g * 16, 16)] = v >> 2
        pltpu.async_copy(ga_hbm.at[idx_v], rows_v, sem0).wait()
        bufs = [mrow0, mrow1]

        def zb(i, _):
            mrow0[0, pl.ds(i * 16, 16)] = jnp.zeros((16,), jnp.float32)
            mrow0[1, pl.ds(i * 16, 16)] = jnp.zeros((16,), jnp.float32)
            mrow1[0, pl.ds(i * 16, 16)] = jnp.zeros((16,), jnp.float32)
            mrow1[1, pl.ds(i * 16, 16)] = jnp.zeros((16,), jnp.float32)
            return 0

        lax.fori_loop(0, VS // 16, zb, 0)
        ones16 = jnp.ones((16,), jnp.float32)
        zeros16 = jnp.zeros((16,), jnp.float32)
        iota16 = lax.iota(jnp.int32, 16)
        tok16 = lax.shift_right_logical(iota16, 3)
        sems = [sem0, sem1]
        pending = [None, None]

        def scat(p, buf, val16):
            row16 = 16 * p + iota16
            q16 = q_v[pl.ds(16 * p, 16)]
            for j in range(GA):
                nb16 = plsc.load_gather(rows_v, [row16, q16 + j])
                plsc.store_scatter(buf, [tok16, nb16], val16)

        for p in range(npair):
            b = p % 2
            if pending[b] is not None:
                pending[b].wait()
                scat(p - 2, bufs[b], zeros16)
            scat(p, bufs[b], ones16)
            pending[b] = pltpu.async_copy(
                bufs[b], mask_hbm.at[pl.ds(wid * tpw + 2 * p, 2)], sems[b])
        for b in range(2):
            if pending[b] is not None:
                pending[b].wait()

    return kfn(kflat, ga_packed)


def kernel(task1_feat, task2_feat, W_g, b_g, W_s, b_s, grapharea, k):
    raw_pad, segmax, lse = _global_head(task1_feat, W_g,
                                        b_g.reshape(1, VG))
    predictions_globals = _shift_logits(raw_pad, lse)
    segids = _top_segments(segmax.transpose(1, 0, 2).reshape(T, NSEG))
    fseg = (jnp.arange(T, dtype=jnp.int32)[:, None] * NSEG + segids).reshape(-1)
    cands = _sc_gather(raw_pad.reshape(T * NSEG, SEGW), fseg)
    kidx = _top8_refine(cands.reshape(T, KTOP * SEGW), segids)
    kidx = kidx + (jnp.asarray(k, dtype=kidx.dtype) - KTOP)
    mask = _sc_mask(kidx.reshape(-1), grapharea.reshape(VG // 4, 4 * GA))
    predictions_senses = _sense_head(task2_feat, W_s, b_s.reshape(1, VS), mask)
    return predictions_globals, predictions_senses
